# baseline (speedup 1.0000x reference)
import jax
import jax.numpy as jnp
from jax import lax
from jax.experimental import pallas as pl
from jax.experimental.pallas import tpu as pltpu
from jax.experimental.pallas import tpu_sc as plsc

VOCAB = 100000
VPAD = 100096
DIM = 64
B = 4096
L = 200
LP = 208
NC = 2
NS = 16
NW = NC * NS
BPW = B // NW
NCHUNK = LP // 16
HALF = LP // 2
DW = DIM // 4

_GDN = lax.GatherDimensionNumbers(
    offset_dims=(), collapsed_slice_dims=(0,), start_index_map=(0,))


def _shuffle(vec, ind):
    return lax.gather(vec, ind.reshape(16, 1), _GDN, (1,),
                      mode=lax.GatherScatterMode.PROMISE_IN_BOUNDS)


def _sc_body(terms_hbm, cnts_hbm, combo_hbm, qtab_hbm, out_hbm,
             t2, c2, cb2, rows2, o2, stab, stabc,
             sem_t0, sem_t1, sem_g0, sem_g1, sem_o0, sem_o1, sem_w, sem_s):
    sem_t = (sem_t0, sem_t1)
    sem_g = (sem_g0, sem_g1)
    sem_o = (sem_o0, sem_o1)
    wid = lax.axis_index("s") * NC + lax.axis_index("c")
    base = wid * BPW

    def issue_tc(r, p):
        pltpu.async_copy(terms_hbm.at[base + r], t2.at[p], sem_t[p])
        pltpu.async_copy(cnts_hbm.at[base + r], c2.at[p], sem_t[p])

    def wait_tc(p):
        pltpu.make_async_copy(terms_hbm.at[0], t2.at[p], sem_t[p]).wait()
        pltpu.make_async_copy(cnts_hbm.at[0], c2.at[p], sem_t[p]).wait()

    def issue_gather(p):
        for h in range(2):
            sl = pl.ds(h * HALF, HALF)
            idx = t2.at[p].at[sl]
            pltpu.async_copy(stab.at[idx], rows2.at[p].at[sl], sem_g[p])
            pltpu.async_copy(stabc.at[idx], cb2.at[p].at[sl], sem_g[p])

    def wait_gather(p):
        pltpu.make_async_copy(qtab_hbm.at[pl.ds(0, LP)], rows2.at[p],
                              sem_g[p]).wait()
        pltpu.make_async_copy(combo_hbm.at[pl.ds(0, LP)], cb2.at[p],
                              sem_g[p]).wait()

    def wait_out(p):
        pltpu.make_async_copy(o2.at[p], out_hbm.at[0], sem_o[p]).wait()

    iota = lax.iota(jnp.int32, 16)
    hi_mask = jnp.full((16,), -65536, jnp.int32)
    ph = iota & 3
    m0, m1, m2 = ph == 0, ph == 1, ph == 2

    issue_tc(0, 0)
    issue_tc(1, 1)
    sid = lax.axis_index("s")
    SLICE = VPAD // NS
    scopy = pltpu.async_copy(qtab_hbm.at[pl.ds(sid * SLICE, SLICE)],
                             stab.at[pl.ds(sid * SLICE, SLICE)], sem_s)
    ccopy = pltpu.async_copy(combo_hbm.at[pl.ds(sid * SLICE, SLICE)],
                             stabc.at[pl.ds(sid * SLICE, SLICE)], sem_w)
    scopy.wait()
    ccopy.wait()
    plsc.subcore_barrier()
    wait_tc(0)
    issue_gather(0)

    def iteration(g, carry):
        for p in (0, 1):
            r = 2 * g + p
            wait_gather(p)
            pcs, scs = [], []
            s_vec = jnp.zeros((16,), jnp.float32)
            for c in range(NCHUNK):
                cb = cb2[p, pl.ds(c * 16, 16)]
                wc = lax.bitcast_convert_type(cb & hi_mask, jnp.float32)
                sc = lax.bitcast_convert_type(
                    lax.shift_left(cb, 16), jnp.float32)
                cc = c2[p, pl.ds(c * 16, 16)].astype(jnp.float32)
                pc = cc * jnp.exp(wc)
                pcs.append(pc)
                scs.append(sc)
                s_vec = s_vec + pc
            for sh in (8, 4, 2, 1):
                s_vec = s_vec + _shuffle(s_vec, iota ^ sh)
            inv = jnp.float32(1.0) / s_vec
            pcs = [pc * inv * sc for pc, sc in zip(pcs, scs)]

            @pl.when(r + 2 < BPW)
            def _():
                issue_tc(r + 2, p)

            @pl.when(r + 1 < BPW)
            def _():
                wait_tc(1 - p)
                issue_gather(1 - p)

            @pl.when(g >= 1)
            def _():
                wait_out(p)

            acc = [jnp.zeros((16,), jnp.float32) for _ in range(4)]
            for c in range(NCHUNK):
                for j in range(16):
                    l = c * 16 + j
                    pb = _shuffle(pcs[c], jnp.full((16,), j, jnp.int32))
                    vi = rows2[p, l, :]
                    for gph in range(4):
                        sh = 24 - 8 * gph
                        x = lax.shift_left(vi, sh) if sh else vi
                        xf = lax.shift_right_arithmetic(x, 24).astype(
                            jnp.float32)
                        acc[gph] = acc[gph] + pb * xf
            for q in range(4):
                pidx = (iota >> 2) + 4 * q
                pg = [_shuffle(a, pidx) for a in acc]
                o2[p, pl.ds(q * 16, 16)] = jnp.where(
                    m0, pg[0], jnp.where(m1, pg[1],
                                         jnp.where(m2, pg[2], pg[3])))
            pltpu.async_copy(o2.at[p], out_hbm.at[base + r], sem_o[p])
        return carry

    lax.fori_loop(0, BPW // 2, iteration, 0)
    wait_out(0)
    wait_out(1)


@jax.jit
def _run(terms_p, cnts_p, combo, qtab):
    mesh = plsc.VectorSubcoreMesh(core_axis_name="c", subcore_axis_name="s")
    kfn = pl.kernel(
        _sc_body,
        out_type=jax.ShapeDtypeStruct((B, DIM), jnp.float32),
        mesh=mesh,
        scratch_types=[
            pltpu.VMEM((2, LP), jnp.int32),
            pltpu.VMEM((2, LP), jnp.int32),
            pltpu.VMEM((2, LP), jnp.int32),
            pltpu.VMEM((2, LP, DW), jnp.int32),
            pltpu.VMEM((2, DIM), jnp.float32),
            pltpu.VMEM_SHARED((VPAD, DW), jnp.int32),
            pltpu.VMEM_SHARED((VPAD,), jnp.int32),
            pltpu.SemaphoreType.DMA,
            pltpu.SemaphoreType.DMA,
            pltpu.SemaphoreType.DMA,
            pltpu.SemaphoreType.DMA,
            pltpu.SemaphoreType.DMA,
            pltpu.SemaphoreType.DMA,
            pltpu.SemaphoreType.DMA,
            pltpu.SemaphoreType.DMA,
        ],
        compiler_params=pltpu.CompilerParams(
            use_tc_tiling_on_sc=False, needs_layout_passes=False),
    )
    return kfn(terms_p, cnts_p, combo, qtab)


def kernel(terms, cnts, weights, table):
    terms_p = jnp.pad(terms, ((0, 0), (0, LP - L)))
    cnts_p = jnp.pad(cnts, ((0, 0), (0, LP - L)))
    absmax = jnp.max(jnp.abs(table), axis=1)
    scale = jnp.maximum(absmax, jnp.float32(1e-30)) / 127.0
    q = jnp.clip(jnp.round(table / scale[:, None]), -127, 127).astype(
        jnp.int8)
    qtab = lax.bitcast_convert_type(q.reshape(VOCAB, DW, 4), jnp.int32)
    qtab = jnp.pad(qtab, ((0, VPAD - VOCAB), (0, 0)))
    wb = lax.bitcast_convert_type(
        weights.reshape(VOCAB).astype(jnp.bfloat16), jnp.uint16)
    sb = lax.bitcast_convert_type(scale.astype(jnp.bfloat16), jnp.uint16)
    combo = lax.bitcast_convert_type(
        (wb.astype(jnp.uint32) << 16) | sb.astype(jnp.uint32), jnp.int32)
    combo = jnp.pad(combo, (0, VPAD - VOCAB))
    return _run(terms_p, cnts_p, combo, qtab)


# speedup vs baseline: 1.7006x; 1.7006x over previous
import jax
import jax.numpy as jnp
from jax import lax
from jax.experimental import pallas as pl
from jax.experimental.pallas import tpu as pltpu
from jax.experimental.pallas import tpu_sc as plsc

VOCAB = 100000
DIM = 64
B = 4096
L = 200
LP = 208
NC = 2
NS = 16
NW = NC * NS
BPW = B // NW
NCHUNK = LP // 16
HALF = LP // 2

_GDN = lax.GatherDimensionNumbers(
    offset_dims=(), collapsed_slice_dims=(0,), start_index_map=(0,))


def _shuffle(vec, ind):
    return lax.gather(vec, ind.reshape(16, 1), _GDN, (1,),
                      mode=lax.GatherScatterMode.PROMISE_IN_BOUNDS)


def _sc_body(terms_hbm, cnts_hbm, w_hbm, table_hbm, out_hbm,
             t2, c2, w2, rows2, o2,
             sem_t0, sem_t1, sem_g0, sem_g1, sem_o0, sem_o1):
    sem_t = (sem_t0, sem_t1)
    sem_g = (sem_g0, sem_g1)
    sem_o = (sem_o0, sem_o1)
    wid = lax.axis_index("s") * NC + lax.axis_index("c")
    base = wid * BPW

    def issue_tc(r, p):
        pltpu.async_copy(terms_hbm.at[base + r], t2.at[p], sem_t[p])
        pltpu.async_copy(cnts_hbm.at[base + r], c2.at[p], sem_t[p])

    def wait_tc(p):
        pltpu.make_async_copy(terms_hbm.at[0], t2.at[p], sem_t[p]).wait()
        pltpu.make_async_copy(cnts_hbm.at[0], c2.at[p], sem_t[p]).wait()

    def issue_gather(p):
        for off, ln in ((0, HALF), (HALF, L - HALF)):
            sl = pl.ds(off, ln)
            idx = t2.at[p].at[sl]
            pltpu.async_copy(w_hbm.at[idx], w2.at[p].at[sl], sem_g[p])
            pltpu.async_copy(table_hbm.at[idx], rows2.at[p].at[sl], sem_g[p])

    def wait_gather(p):
        pltpu.make_async_copy(table_hbm.at[pl.ds(0, L)],
                              rows2.at[p].at[pl.ds(0, L)], sem_g[p]).wait()
        pltpu.make_async_copy(w_hbm.at[pl.ds(0, L)],
                              w2.at[p].at[pl.ds(0, L)], sem_g[p]).wait()

    def wait_out(p):
        pltpu.make_async_copy(o2.at[p], out_hbm.at[0], sem_o[p]).wait()

    iota = lax.iota(jnp.int32, 16)
    parity_odd = (iota & 1) == 1
    hi_mask = jnp.full((16,), -65536, jnp.int32)

    issue_tc(0, 0)
    issue_tc(1, 1)
    zb = jnp.zeros((32,), jnp.bfloat16)
    zf = jnp.zeros((16,), jnp.float32)
    for p in (0, 1):
        for l in range(L, LP):
            for h in range(2):
                rows2[p, l, pl.ds(h * 32, 32)] = zb
        w2[p, pl.ds(192, 16)] = zf
    wait_tc(0)
    issue_gather(0)

    def iteration(g, carry):
        for p in (0, 1):
            r = 2 * g + p
            wait_gather(p)
            pcs = []
            s_vec = jnp.zeros((16,), jnp.float32)
            for c in range(NCHUNK):
                wc = w2[p, pl.ds(c * 16, 16)]
                cc = c2[p, pl.ds(c * 16, 16)].astype(jnp.float32)
                pc = cc * jnp.exp(wc)
                pcs.append(pc)
                s_vec = s_vec + pc
            for sh in (8, 4, 2, 1):
                s_vec = s_vec + _shuffle(s_vec, iota ^ sh)
            inv = jnp.float32(1.0) / s_vec
            pcs = [pc * inv for pc in pcs]

            @pl.when(r + 2 < BPW)
            def _():
                issue_tc(r + 2, p)

            @pl.when(r + 1 < BPW)
            def _():
                wait_tc(1 - p)
                issue_gather(1 - p)

            @pl.when(g >= 1)
            def _():
                wait_out(p)

            acc = [jnp.zeros((16,), jnp.float32) for _ in range(4)]
            for c in range(NCHUNK):
                for j in range(16):
                    l = c * 16 + j
                    pb = _shuffle(pcs[c], jnp.full((16,), j, jnp.int32))
                    for h in range(2):
                        vb = rows2[p, l, pl.ds(h * 32, 32)]
                        vi = plsc.bitcast(vb, jnp.int32)
                        lo = lax.bitcast_convert_type(
                            lax.shift_left(vi, 16), jnp.float32)
                        hi = lax.bitcast_convert_type(vi & hi_mask,
                                                      jnp.float32)
                        acc[2 * h] = acc[2 * h] + pb * lo
                        acc[2 * h + 1] = acc[2 * h + 1] + pb * hi
            for q in range(4):
                h = q // 2
                pidx = (iota >> 1) + (q % 2) * 8
                ev = _shuffle(acc[2 * h], pidx)
                od = _shuffle(acc[2 * h + 1], pidx)
                o2[p, pl.ds(q * 16, 16)] = jnp.where(parity_odd, od, ev)
            pltpu.async_copy(o2.at[p], out_hbm.at[base + r], sem_o[p])
        return carry

    lax.fori_loop(0, BPW // 2, iteration, 0)
    wait_out(0)
    wait_out(1)


@jax.jit
def _run(terms_p, cnts_p, w_flat, table_b):
    mesh = plsc.VectorSubcoreMesh(core_axis_name="c", subcore_axis_name="s")
    kfn = pl.kernel(
        _sc_body,
        out_type=jax.ShapeDtypeStruct((B, DIM), jnp.float32),
        mesh=mesh,
        scratch_types=[
            pltpu.VMEM((2, LP), jnp.int32),
            pltpu.VMEM((2, LP), jnp.int32),
            pltpu.VMEM((2, LP), jnp.float32),
            pltpu.VMEM((2, LP, DIM), jnp.bfloat16),
            pltpu.VMEM((2, DIM), jnp.float32),
            pltpu.SemaphoreType.DMA,
            pltpu.SemaphoreType.DMA,
            pltpu.SemaphoreType.DMA,
            pltpu.SemaphoreType.DMA,
            pltpu.SemaphoreType.DMA,
            pltpu.SemaphoreType.DMA,
        ],
        compiler_params=pltpu.CompilerParams(
            use_tc_tiling_on_sc=False, needs_layout_passes=False),
    )
    return kfn(terms_p, cnts_p, w_flat, table_b)


def kernel(terms, cnts, weights, table):
    terms_p = jnp.pad(terms, ((0, 0), (0, LP - L)))
    cnts_p = jnp.pad(cnts, ((0, 0), (0, LP - L)))
    w_flat = weights.reshape(VOCAB)
    return _run(terms_p, cnts_p, w_flat, table.astype(jnp.bfloat16))
